# baseline (speedup 1.0000x reference)
import functools

import jax
import jax.numpy as jnp
import numpy as np
from jax import lax
from jax.experimental import pallas as pl
from jax.experimental.pallas import tpu as pltpu
from jax.experimental.pallas import tpu_sc as plsc

B, T, N = 8, 12, 10000
ROWS = B * T * N
NC, NS = 2, 16
NW = NC * NS
RPW = ROWS // NW
MASK_RATIO = 0.15


def _threefry2x32(k0, k1, x0, x1):
    rotations = ((13, 15, 26, 6), (17, 29, 16, 24))
    k0 = np.uint32(k0)
    k1 = np.uint32(k1)
    ks = (k0, k1, np.uint32(k0 ^ k1 ^ np.uint32(0x1BD11BDA)))
    x0 = (x0 + k0).astype(np.uint32)
    x1 = (x1 + k1).astype(np.uint32)

    def rot(x, d):
        return ((x << np.uint32(d)) | (x >> np.uint32(32 - d))).astype(np.uint32)

    for i in range(5):
        for r in rotations[i % 2]:
            x0 = (x0 + x1).astype(np.uint32)
            x1 = rot(x1, r) ^ x0
        x0 = (x0 + ks[(i + 1) % 3]).astype(np.uint32)
        x1 = (x1 + ks[(i + 2) % 3] + np.uint32(i + 1)).astype(np.uint32)
    return x0, x1


@functools.lru_cache(maxsize=None)
def _mask_consts():
    def bits32(kk, n):
        o0, o1 = _threefry2x32(
            kk[0], kk[1], np.zeros(n, np.uint32), np.arange(n, dtype=np.uint32))
        return o0 ^ o1

    def mask_from(kk, shape):
        bits = bits32(kk, int(np.prod(shape)))
        f = ((bits >> np.uint32(9)) | np.uint32(0x3F800000)).view(np.float32)
        u = np.maximum(np.float32(0.0), f - np.float32(1.0))
        return (u < np.float32(MASK_RATIO)).reshape(shape)

    o0, o1 = _threefry2x32(0, 42, np.zeros(2, np.uint32), np.arange(2, dtype=np.uint32))
    ka, kb = (o0[0], o1[0]), (o0[1], o1[1])
    return mask_from(ka, (B, T, N)), mask_from(kb, (B, T, N))


@functools.lru_cache(maxsize=None)
def _plan(which: int, C: int, D: int):
    m = _mask_consts()[which].reshape(-1)
    nch = RPW // C
    per = m.reshape(NW, nch, C)
    kmax = int(per.sum(axis=2).max())
    kpad = -(-kmax // 16) * 16
    idx = np.full((NW, nch * kpad), C * D, dtype=np.int32)
    for w in range(NW):
        for g in range(nch):
            rows = np.nonzero(per[w, g])[0].astype(np.int32)
            idx[w, g * kpad: g * kpad + rows.size] = rows * D
    return idx, kpad, nch


def _make_masked_copy(D: int, C: int, kpad: int, nch: int):
    CD = C * D
    Lw = nch * kpad
    mesh = plsc.VectorSubcoreMesh(core_axis_name="c", subcore_axis_name="s")

    @functools.partial(
        pl.kernel,
        out_type=jax.ShapeDtypeStruct((ROWS * D,), jnp.float32),
        mesh=mesh,
        compiler_params=pltpu.CompilerParams(needs_layout_passes=False),
        scratch_types=[
            pltpu.VMEM((CD + 64,), jnp.float32),
            pltpu.VMEM((Lw,), jnp.int32),
            pltpu.VMEM((D, 16), jnp.float32),
        ],
    )
    def kern(x_hbm, idx_hbm, tok_hbm, out_hbm, buf, idx_v, tok_v):
        wid = lax.axis_index("s") * NC + lax.axis_index("c")
        base = wid * (RPW * D)
        pltpu.sync_copy(idx_hbm.at[wid], idx_v)
        pltpu.sync_copy(tok_hbm, tok_v)

        def body(g, carry):
            start = base + g * CD
            pltpu.sync_copy(x_hbm.at[pl.ds(start, CD)], buf.at[pl.ds(0, CD)])
            for k in range(kpad // 16):
                bases = idx_v[pl.ds(g * kpad + k * 16, 16)]
                for c in range(D):
                    plsc.store_scatter(buf, [bases + c], tok_v[c])
            pltpu.sync_copy(buf.at[pl.ds(0, CD)], out_hbm.at[pl.ds(start, CD)])
            return carry

        lax.fori_loop(0, nch, body, 0)

    return kern


def kernel(feat0, feat1, mask_token0, mask_token1):
    m0, m1 = _mask_consts()
    idx0, kpad0, nch0 = _plan(0, 500, 64)
    idx1, kpad1, nch1 = _plan(1, 1000, 32)

    k0 = _make_masked_copy(64, 500, kpad0, nch0)
    k1 = _make_masked_copy(32, 1000, kpad1, nch1)

    tok0 = jnp.broadcast_to(mask_token0.reshape(64, 1), (64, 16))
    tok1 = jnp.broadcast_to(mask_token1.reshape(32, 1), (32, 16))

    out0 = k0(feat0.reshape(-1), jnp.asarray(idx0), tok0)
    out1 = k1(feat1.reshape(-1), jnp.asarray(idx1), tok1)

    return (
        out0.reshape(B, T, N, 64),
        out1.reshape(B, T, N, 32),
        jnp.asarray(m0),
        jnp.asarray(m1),
    )


# speedup vs baseline: 1.0052x; 1.0052x over previous
import functools

import jax
import jax.numpy as jnp
import numpy as np
from jax import lax
from jax.experimental import pallas as pl
from jax.experimental.pallas import tpu as pltpu
from jax.experimental.pallas import tpu_sc as plsc

B, T, N = 8, 12, 10000
ROWS = B * T * N
NC, NS = 2, 16
NW = NC * NS
RPW = ROWS // NW
MASK_RATIO = 0.15


def _threefry2x32(k0, k1, x0, x1):
    rotations = ((13, 15, 26, 6), (17, 29, 16, 24))
    k0 = np.uint32(k0)
    k1 = np.uint32(k1)
    ks = (k0, k1, np.uint32(k0 ^ k1 ^ np.uint32(0x1BD11BDA)))
    x0 = (x0 + k0).astype(np.uint32)
    x1 = (x1 + k1).astype(np.uint32)

    def rot(x, d):
        return ((x << np.uint32(d)) | (x >> np.uint32(32 - d))).astype(np.uint32)

    for i in range(5):
        for r in rotations[i % 2]:
            x0 = (x0 + x1).astype(np.uint32)
            x1 = rot(x1, r) ^ x0
        x0 = (x0 + ks[(i + 1) % 3]).astype(np.uint32)
        x1 = (x1 + ks[(i + 2) % 3] + np.uint32(i + 1)).astype(np.uint32)
    return x0, x1


@functools.lru_cache(maxsize=None)
def _mask_consts():
    def bits32(kk, n):
        o0, o1 = _threefry2x32(
            kk[0], kk[1], np.zeros(n, np.uint32), np.arange(n, dtype=np.uint32))
        return o0 ^ o1

    def mask_from(kk, shape):
        bits = bits32(kk, int(np.prod(shape)))
        f = ((bits >> np.uint32(9)) | np.uint32(0x3F800000)).view(np.float32)
        u = np.maximum(np.float32(0.0), f - np.float32(1.0))
        return (u < np.float32(MASK_RATIO)).reshape(shape)

    o0, o1 = _threefry2x32(0, 42, np.zeros(2, np.uint32), np.arange(2, dtype=np.uint32))
    ka, kb = (o0[0], o1[0]), (o0[1], o1[1])
    return mask_from(ka, (B, T, N)), mask_from(kb, (B, T, N))


@functools.lru_cache(maxsize=None)
def _plan(which: int, C: int, D: int):
    m = _mask_consts()[which].reshape(-1)
    nch = RPW // C
    per = m.reshape(NW, nch, C)
    kmax = int(per.sum(axis=2).max())
    kpad = -(-kmax // 16) * 16
    idx = np.full((NW, nch * kpad), C * D, dtype=np.int32)
    for w in range(NW):
        for g in range(nch):
            rows = np.nonzero(per[w, g])[0].astype(np.int32)
            idx[w, g * kpad: g * kpad + rows.size] = rows * D
    return idx, kpad, nch


NBUF = 3


def _make_masked_copy(D: int, C: int, kpad: int, nch: int):
    CD = C * D
    Lw = nch * kpad
    mesh = plsc.VectorSubcoreMesh(core_axis_name="c", subcore_axis_name="s")
    assert nch >= NBUF

    assert nch % NBUF == 0

    @functools.partial(
        pl.kernel,
        out_type=jax.ShapeDtypeStruct((ROWS * D,), jnp.float32),
        mesh=mesh,
        compiler_params=pltpu.CompilerParams(needs_layout_passes=False),
        scratch_types=(
            [pltpu.VMEM((CD + 64,), jnp.float32)] * NBUF
            + [pltpu.VMEM((Lw,), jnp.int32), pltpu.VMEM((D, 16), jnp.float32)]
            + [pltpu.SemaphoreType.DMA] * (2 * NBUF)
        ),
    )
    def kern(x_hbm, idx_hbm, tok_hbm, out_hbm, *scratch):
        bufs = scratch[:NBUF]
        idx_v, tok_v = scratch[NBUF], scratch[NBUF + 1]
        sins = scratch[NBUF + 2: 2 * NBUF + 2]
        souts = scratch[2 * NBUF + 2:]
        wid = lax.axis_index("s") * NC + lax.axis_index("c")
        base = wid * (RPW * D)
        pltpu.sync_copy(idx_hbm.at[wid], idx_v)
        pltpu.sync_copy(tok_hbm, tok_v)

        def in_copy(g, b):
            return pltpu.make_async_copy(
                x_hbm.at[pl.ds(base + g * CD, CD)], bufs[b].at[pl.ds(0, CD)],
                sins[b])

        def out_copy(g, b):
            return pltpu.make_async_copy(
                bufs[b].at[pl.ds(0, CD)], out_hbm.at[pl.ds(base + g * CD, CD)],
                souts[b])

        for g0 in range(NBUF - 1):
            in_copy(g0, g0).start()

        def body(go, carry):
            for j in range(NBUF):
                g = go * NBUF + j
                in_copy(g, j).wait()
                bases = [idx_v[pl.ds(g * kpad + k * 16, 16)]
                         for k in range(kpad // 16)]
                for c in range(D):
                    tv = tok_v[c]
                    for bs in bases:
                        plsc.store_scatter(bufs[j], [bs + c], tv)

                jn = (j + NBUF - 1) % NBUF

                @pl.when((g >= 1) & (g + NBUF - 1 < nch))
                def _():
                    out_copy(g - 1, jn).wait()

                @pl.when(g + NBUF - 1 < nch)
                def _():
                    in_copy(g + NBUF - 1, jn).start()

                out_copy(g, j).start()
            return carry

        lax.fori_loop(0, nch // NBUF, body, 0)
        for g in range(nch - NBUF, nch):
            out_copy(g, g % NBUF).wait()

    return kern


def kernel(feat0, feat1, mask_token0, mask_token1):
    m0, m1 = _mask_consts()
    idx0, kpad0, nch0 = _plan(0, 500, 64)
    idx1, kpad1, nch1 = _plan(1, 1000, 32)

    k0 = _make_masked_copy(64, 500, kpad0, nch0)
    k1 = _make_masked_copy(32, 1000, kpad1, nch1)

    tok0 = jnp.broadcast_to(mask_token0.reshape(64, 1), (64, 16))
    tok1 = jnp.broadcast_to(mask_token1.reshape(32, 1), (32, 16))

    out0 = k0(feat0.reshape(-1), jnp.asarray(idx0), tok0)
    out1 = k1(feat1.reshape(-1), jnp.asarray(idx1), tok1)

    return (
        out0.reshape(B, T, N, 64),
        out1.reshape(B, T, N, 32),
        jnp.asarray(m0),
        jnp.asarray(m1),
    )


# speedup vs baseline: 3.6228x; 3.6041x over previous
import functools

import jax
import jax.numpy as jnp
import numpy as np
from jax import lax
from jax.experimental import pallas as pl
from jax.experimental.pallas import tpu as pltpu
from jax.experimental.pallas import tpu_sc as plsc

B, T, N = 8, 12, 10000
NBT = B * T
NC, NS = 2, 16
NW = NC * NS
NJ = N // 16


def _threefry2x32(k0, k1, x0, x1):
    rotations = ((13, 15, 26, 6), (17, 29, 16, 24))
    k0 = np.uint32(k0)
    k1 = np.uint32(k1)
    ks = (k0, k1, np.uint32(k0 ^ k1 ^ np.uint32(0x1BD11BDA)))
    x0 = (x0 + k0).astype(np.uint32)
    x1 = (x1 + k1).astype(np.uint32)

    def rot(x, d):
        return ((x << np.uint32(d)) | (x >> np.uint32(32 - d))).astype(np.uint32)

    for i in range(5):
        for r in rotations[i % 2]:
            x0 = (x0 + x1).astype(np.uint32)
            x1 = rot(x1, r) ^ x0
        x0 = (x0 + ks[(i + 1) % 3]).astype(np.uint32)
        x1 = (x1 + ks[(i + 2) % 3] + np.uint32(i + 1)).astype(np.uint32)
    return x0, x1


@functools.lru_cache(maxsize=None)
def _mask_consts():
    def bits32(kk, n):
        o0, o1 = _threefry2x32(
            kk[0], kk[1], np.zeros(n, np.uint32), np.arange(n, dtype=np.uint32))
        return o0 ^ o1

    def mask_from(kk, shape):
        bits = bits32(kk, int(np.prod(shape)))
        f = ((bits >> np.uint32(9)) | np.uint32(0x3F800000)).view(np.float32)
        u = np.maximum(np.float32(0.0), f - np.float32(1.0))
        return (u < np.float32(0.15)).reshape(shape)

    o0, o1 = _threefry2x32(0, 42, np.zeros(2, np.uint32), np.arange(2, dtype=np.uint32))
    ka, kb = (o0[0], o1[0]), (o0[1], o1[1])
    return mask_from(ka, (B, T, N)), mask_from(kb, (B, T, N))


def _make_sc_select(D: int):
    ngrp = D // 8
    nslab = NBT * ngrp
    spw = nslab // NW
    nbt_pw = spw // ngrp
    mesh = plsc.VectorSubcoreMesh(core_axis_name="c", subcore_axis_name="s")

    @functools.partial(
        pl.kernel,
        out_type=jax.ShapeDtypeStruct((nslab, 8, N), jnp.float32),
        mesh=mesh,
        compiler_params=pltpu.CompilerParams(
            needs_layout_passes=False, use_tc_tiling_on_sc=True),
        scratch_types=[
            pltpu.VMEM((8, N), jnp.float32),
            pltpu.VMEM((N,), jnp.int32),
            pltpu.VMEM((16 * D,), jnp.float32),
        ],
    )
    def kern(x_hbm, mask_hbm, tok_hbm, out_hbm, buf, mask_v, tok_v):
        wid = lax.axis_index("s") * NC + lax.axis_index("c")
        pltpu.sync_copy(tok_hbm, tok_v)

        def bt_body(i, carry):
            bt = wid * nbt_pw + i
            pltpu.sync_copy(mask_hbm.at[bt], mask_v)

            def g_body(g, carry2):
                s = bt * ngrp + g
                pltpu.sync_copy(x_hbm.at[s], buf)
                toks = [tok_v[pl.ds((g * 8 + r) * 16, 16)] for r in range(8)]

                def j_body(j, carry3):
                    sl = pl.ds(j * 16, 16)
                    mb = mask_v[sl] != 0
                    for r in range(8):
                        buf[r, sl] = jnp.where(mb, toks[r], buf[r, sl])
                    return carry3

                lax.fori_loop(0, NJ, j_body, 0)
                pltpu.sync_copy(buf, out_hbm.at[s])
                return carry2

            lax.fori_loop(0, ngrp, g_body, 0)
            return carry

        lax.fori_loop(0, nbt_pw, bt_body, 0)

    return kern


def _make_tc_select(D: int):
    ngrp = D // 8
    nslab = NBT * ngrp

    def body(x_ref, m_ref, tok_ref, o_ref):
        m = m_ref[...] != 0
        t = tok_ref[...][0, 0][None, :, None]
        o_ref[...] = jnp.where(m, t, x_ref[...])

    return pl.pallas_call(
        body,
        grid=(nslab,),
        in_specs=[
            pl.BlockSpec((1, 8, N), lambda s: (s, 0, 0)),
            pl.BlockSpec((1, 1, N), lambda s: (s // ngrp, 0, 0)),
            pl.BlockSpec((1, 1, 8), lambda s: (s, 0, 0)),
        ],
        out_specs=pl.BlockSpec((1, 8, N), lambda s: (s, 0, 0)),
        out_shape=jax.ShapeDtypeStruct((nslab, 8, N), jnp.float32),
    )


def kernel(feat0, feat1, mask_token0, mask_token1):
    m0, m1 = _mask_consts()

    xt0 = feat0.transpose(0, 1, 3, 2).reshape(NBT * 8, 8, N)
    xt1 = feat1.transpose(0, 1, 3, 2).reshape(NBT * 4, 8, N)

    m0_i = jnp.asarray(m0.reshape(NBT, N).astype(np.int32))
    m1_i = jnp.asarray(m1.reshape(NBT, N).astype(np.int32))

    t0 = mask_token0.reshape(-1)
    t1 = mask_token1.reshape(-1)
    tok0_tc = jnp.broadcast_to(t0.reshape(1, 8, 8), (NBT, 8, 8)).reshape(NBT * 8, 8)
    tok1_sc = jnp.broadcast_to(t1.reshape(32, 1), (32, 16)).reshape(512)

    yt0 = _make_tc_select(64)(
        xt0, m0_i.reshape(NBT, 1, N), tok0_tc.reshape(NBT * 8, 1, 8))
    yt1 = _make_sc_select(32)(xt1, m1_i, tok1_sc)

    out0 = yt0.reshape(B, T, 64, N).transpose(0, 1, 3, 2)
    out1 = yt1.reshape(B, T, 32, N).transpose(0, 1, 3, 2)
    return (out0, out1, jnp.asarray(m0), jnp.asarray(m1))


# speedup vs baseline: 7.7064x; 2.1272x over previous
import functools

import jax
import jax.numpy as jnp
import numpy as np
from jax import lax
from jax.experimental import pallas as pl
from jax.experimental.pallas import tpu as pltpu
from jax.experimental.pallas import tpu_sc as plsc

B, T, N = 8, 12, 10000
NBT = B * T
NC, NS = 2, 16
NW = NC * NS
NJ = N // 16


def _threefry2x32(k0, k1, x0, x1):
    rotations = ((13, 15, 26, 6), (17, 29, 16, 24))
    k0 = np.uint32(k0)
    k1 = np.uint32(k1)
    ks = (k0, k1, np.uint32(k0 ^ k1 ^ np.uint32(0x1BD11BDA)))
    x0 = (x0 + k0).astype(np.uint32)
    x1 = (x1 + k1).astype(np.uint32)

    def rot(x, d):
        return ((x << np.uint32(d)) | (x >> np.uint32(32 - d))).astype(np.uint32)

    for i in range(5):
        for r in rotations[i % 2]:
            x0 = (x0 + x1).astype(np.uint32)
            x1 = rot(x1, r) ^ x0
        x0 = (x0 + ks[(i + 1) % 3]).astype(np.uint32)
        x1 = (x1 + ks[(i + 2) % 3] + np.uint32(i + 1)).astype(np.uint32)
    return x0, x1


@functools.lru_cache(maxsize=None)
def _mask_consts():
    def bits32(kk, n):
        o0, o1 = _threefry2x32(
            kk[0], kk[1], np.zeros(n, np.uint32), np.arange(n, dtype=np.uint32))
        return o0 ^ o1

    def mask_from(kk, shape):
        bits = bits32(kk, int(np.prod(shape)))
        f = ((bits >> np.uint32(9)) | np.uint32(0x3F800000)).view(np.float32)
        u = np.maximum(np.float32(0.0), f - np.float32(1.0))
        return (u < np.float32(0.15)).reshape(shape)

    o0, o1 = _threefry2x32(0, 42, np.zeros(2, np.uint32), np.arange(2, dtype=np.uint32))
    ka, kb = (o0[0], o1[0]), (o0[1], o1[1])
    return mask_from(ka, (B, T, N)), mask_from(kb, (B, T, N))


def _make_sc_select(D: int):
    ngrp = D // 8
    nslab = NBT * ngrp
    spw = nslab // NW
    nbt_pw = spw // ngrp
    mesh = plsc.VectorSubcoreMesh(core_axis_name="c", subcore_axis_name="s")

    @functools.partial(
        pl.kernel,
        out_type=jax.ShapeDtypeStruct((nslab, 8, N), jnp.float32),
        mesh=mesh,
        compiler_params=pltpu.CompilerParams(
            needs_layout_passes=False, use_tc_tiling_on_sc=True),
        scratch_types=[
            pltpu.VMEM((8, N), jnp.float32),
            pltpu.VMEM((N,), jnp.int32),
            pltpu.VMEM((16 * D,), jnp.float32),
        ],
    )
    def kern(x_hbm, mask_hbm, tok_hbm, out_hbm, buf, mask_v, tok_v):
        wid = lax.axis_index("s") * NC + lax.axis_index("c")
        pltpu.sync_copy(tok_hbm, tok_v)

        def bt_body(i, carry):
            bt = wid * nbt_pw + i
            pltpu.sync_copy(mask_hbm.at[bt], mask_v)

            def g_body(g, carry2):
                s = bt * ngrp + g
                pltpu.sync_copy(x_hbm.at[s], buf)
                toks = [tok_v[pl.ds((g * 8 + r) * 16, 16)] for r in range(8)]

                def j_body(j, carry3):
                    sl = pl.ds(j * 16, 16)
                    mb = mask_v[sl] != 0
                    for r in range(8):
                        buf[r, sl] = jnp.where(mb, toks[r], buf[r, sl])
                    return carry3

                lax.fori_loop(0, NJ, j_body, 0)
                pltpu.sync_copy(buf, out_hbm.at[s])
                return carry2

            lax.fori_loop(0, ngrp, g_body, 0)
            return carry

        lax.fori_loop(0, nbt_pw, bt_body, 0)

    return kern


def _make_tc_select(D: int):
    ngrp = D // 8

    def body(x_ref, m_ref, tok_ref, o_ref):
        m = m_ref[...] != 0
        o_ref[...] = jnp.where(m, tok_ref[...], x_ref[...])

    return pl.pallas_call(
        body,
        grid=(NBT,),
        in_specs=[
            pl.BlockSpec((1, ngrp, 8, N), lambda i: (i, 0, 0, 0)),
            pl.BlockSpec((1, 1, 1, N), lambda i: (i, 0, 0, 0)),
            pl.BlockSpec((1, ngrp, 8, 1), lambda i: (0, 0, 0, 0)),
        ],
        out_specs=pl.BlockSpec((1, ngrp, 8, N), lambda i: (i, 0, 0, 0)),
        out_shape=jax.ShapeDtypeStruct((NBT, ngrp, 8, N), jnp.float32),
    )


def kernel(feat0, feat1, mask_token0, mask_token1):
    m0, m1 = _mask_consts()

    xt0 = feat0.transpose(0, 1, 3, 2).reshape(NBT * 8, 8, N)
    xt1 = feat1.transpose(0, 1, 3, 2).reshape(NBT * 4, 8, N)

    m0_i = jnp.asarray(m0.reshape(NBT, N).astype(np.int32))
    m1_i = jnp.asarray(m1.reshape(NBT, N).astype(np.int32))

    t0 = mask_token0.reshape(-1)
    t1 = mask_token1.reshape(-1)
    tok0_tc = t0.reshape(1, 8, 8, 1)
    tok1_sc = jnp.broadcast_to(t1.reshape(32, 1), (32, 16)).reshape(512)

    yt0 = _make_tc_select(64)(
        xt0.reshape(NBT, 8, 8, N), m0_i.reshape(NBT, 1, 1, N), tok0_tc)
    yt1 = _make_sc_select(32)(xt1, m1_i, tok1_sc)

    out0 = yt0.reshape(B, T, 64, N).transpose(0, 1, 3, 2)
    out1 = yt1.reshape(B, T, 32, N).transpose(0, 1, 3, 2)
    return (out0, out1, jnp.asarray(m0), jnp.asarray(m1))
